# baseline (speedup 1.0000x reference)
import functools

import jax
import jax.numpy as jnp
from jax import lax
from jax.experimental import pallas as pl
from jax.experimental.pallas import tpu as pltpu
from jax.experimental.pallas import tpu_sc as plsc

_MM_BLK = 8000
_LANES = 128


def _mm_body(x_ref, w_ref, o_ref):
    o_ref[...] = jnp.dot(x_ref[...], w_ref[...],
                         preferred_element_type=jnp.float32)


def _project_table(table, W):
    num, dim = table.shape
    w128 = jnp.concatenate(
        [W.T, jnp.zeros((dim, _LANES - W.shape[0]), jnp.float32)], axis=1)
    return pl.pallas_call(
        _mm_body,
        grid=(num // _MM_BLK,),
        in_specs=[
            pl.BlockSpec((_MM_BLK, dim), lambda i: (i, 0)),
            pl.BlockSpec((dim, _LANES), lambda i: (0, 0)),
        ],
        out_specs=pl.BlockSpec((_MM_BLK, _LANES), lambda i: (i, 0)),
        out_shape=jax.ShapeDtypeStruct((num, _LANES), jnp.float32),
    )(table, w128)


def _make_gather(n_flat, dim, chunk):
    info = plsc.get_sparse_core_info()
    nw = info.num_cores * info.num_subcores
    per_w = n_flat // nw
    n_chunks = per_w // chunk
    assert n_chunks % 2 == 0
    n_outer = n_chunks // 2
    mesh = plsc.VectorSubcoreMesh(core_axis_name="c", subcore_axis_name="s")

    @functools.partial(
        pl.kernel,
        mesh=mesh,
        out_type=jax.ShapeDtypeStruct((n_flat, dim), jnp.float32),
        scratch_types=[
            pltpu.VMEM((per_w,), jnp.int32),
            pltpu.VMEM((chunk, _LANES), jnp.float32),
            pltpu.VMEM((chunk, _LANES), jnp.float32),
            pltpu.SemaphoreType.DMA,
            pltpu.SemaphoreType.DMA,
            pltpu.SemaphoreType.DMA,
            pltpu.SemaphoreType.DMA,
        ],
        compiler_params=pltpu.CompilerParams(use_tc_tiling_on_sc=False),
    )
    def gather(tab_hbm, idx_hbm, out_hbm, idx_v, rows0, rows1,
               gsem0, gsem1, osem0, osem1):
        rows = (rows0, rows1)
        gsem = (gsem0, gsem1)
        osem = (osem0, osem1)
        wid = lax.axis_index("s") * info.num_cores + lax.axis_index("c")
        base0 = wid * per_w
        pltpu.sync_copy(idx_hbm.at[pl.ds(base0, per_w)], idx_v)

        def fire(g, b):
            pltpu.async_copy(
                tab_hbm.at[idx_v.at[pl.ds(g * chunk, chunk)]], rows[b],
                gsem[b])

        def wait_gather(g, b):
            pltpu.make_async_copy(
                tab_hbm.at[idx_v.at[pl.ds(g * chunk, chunk)]], rows[b],
                gsem[b]).wait()

        def put(g, b):
            pltpu.async_copy(
                rows[b].at[:, pl.ds(0, dim)],
                out_hbm.at[pl.ds(base0 + g * chunk, chunk)], osem[b])

        def wait_put(g, b):
            pltpu.make_async_copy(
                rows[b].at[:, pl.ds(0, dim)],
                out_hbm.at[pl.ds(base0 + g * chunk, chunk)], osem[b]).wait()

        fire(0, 0)

        def body(i, carry):
            g = i * 2
            wait_gather(g, 0)
            put(g, 0)
            fire(g + 1, 1)
            wait_gather(g + 1, 1)
            put(g + 1, 1)
            wait_put(g, 0)

            @pl.when(i + 1 < n_outer)
            def _():
                fire(g + 2, 0)

            wait_put(g + 1, 1)
            return carry

        lax.fori_loop(0, n_outer, body, 0)

    return gather


def kernel(indexes, table, W):
    b, l = indexes.shape
    num, dim = table.shape
    P128 = _project_table(table, W)
    idx_flat = indexes.reshape(-1).astype(jnp.int32)
    out_flat = _make_gather(b * l, dim, 400)(P128, idx_flat)
    return out_flat.reshape(b, l, dim)


# speedup vs baseline: 1.4607x; 1.4607x over previous
import functools

import jax
import jax.numpy as jnp
from jax import lax
from jax.experimental import pallas as pl
from jax.experimental.pallas import tpu as pltpu
from jax.experimental.pallas import tpu_sc as plsc

_MM_BLK = 8000
_LANES = 128


def _mm_body(x_ref, w_ref, o_ref):
    o_ref[...] = jnp.dot(x_ref[...], w_ref[...],
                         preferred_element_type=jnp.float32)


def _project_table(table, W):
    num, dim = table.shape
    w128 = jnp.concatenate(
        [W.T, jnp.zeros((dim, _LANES - W.shape[0]), jnp.float32)], axis=1)
    return pl.pallas_call(
        _mm_body,
        grid=(num // _MM_BLK,),
        in_specs=[
            pl.BlockSpec((_MM_BLK, dim), lambda i: (i, 0)),
            pl.BlockSpec((dim, _LANES), lambda i: (0, 0)),
        ],
        out_specs=pl.BlockSpec((_MM_BLK, _LANES), lambda i: (i, 0)),
        out_shape=jax.ShapeDtypeStruct((num, _LANES), jnp.float32),
    )(table, w128)


def _make_gather(b, l, dim, nb):
    info = plsc.get_sparse_core_info()
    nw = info.num_cores * info.num_subcores
    rows_w = b // nw
    per_w = rows_w * l
    chunk = nb * l
    n_chunks = rows_w // nb
    assert rows_w % nb == 0 and n_chunks % 2 == 0
    n_outer = n_chunks // 2
    mesh = plsc.VectorSubcoreMesh(core_axis_name="c", subcore_axis_name="s")

    @functools.partial(
        pl.kernel,
        mesh=mesh,
        out_type=jax.ShapeDtypeStruct((b, l, dim), jnp.float32),
        scratch_types=[
            pltpu.VMEM((per_w,), jnp.int32),
            pltpu.VMEM((chunk, _LANES), jnp.float32),
            pltpu.VMEM((chunk, _LANES), jnp.float32),
            pltpu.SemaphoreType.DMA,
            pltpu.SemaphoreType.DMA,
            pltpu.SemaphoreType.DMA,
            pltpu.SemaphoreType.DMA,
        ],
        compiler_params=pltpu.CompilerParams(use_tc_tiling_on_sc=False),
    )
    def gather(tab_hbm, idx_hbm, out_hbm, idx_v, rows0, rows1,
               gsem0, gsem1, osem0, osem1):
        rows = (rows0, rows1)
        gsem = (gsem0, gsem1)
        osem = (osem0, osem1)
        wid = lax.axis_index("s") * info.num_cores + lax.axis_index("c")
        base0 = wid * per_w
        row0 = wid * rows_w
        pltpu.sync_copy(idx_hbm.at[pl.ds(base0, per_w)], idx_v)

        def fire(g, b_):
            pltpu.async_copy(
                tab_hbm.at[idx_v.at[pl.ds(g * chunk, chunk)]], rows[b_],
                gsem[b_])

        def wait_gather(g, b_):
            pltpu.make_async_copy(
                tab_hbm.at[idx_v.at[pl.ds(g * chunk, chunk)]], rows[b_],
                gsem[b_]).wait()

        def put(g, b_):
            for k in range(nb):
                pltpu.async_copy(
                    rows[b_].at[pl.ds(k * l, l), pl.ds(0, dim)],
                    out_hbm.at[row0 + g * nb + k], osem[b_])

        def wait_put(g, b_):
            for k in range(nb):
                pltpu.make_async_copy(
                    rows[b_].at[pl.ds(k * l, l), pl.ds(0, dim)],
                    out_hbm.at[row0 + g * nb + k], osem[b_]).wait()

        fire(0, 0)

        def body(i, carry):
            g = i * 2
            wait_gather(g, 0)
            put(g, 0)
            fire(g + 1, 1)
            wait_gather(g + 1, 1)
            put(g + 1, 1)
            wait_put(g, 0)

            @pl.when(i + 1 < n_outer)
            def _():
                fire(g + 2, 0)

            wait_put(g + 1, 1)
            return carry

        lax.fori_loop(0, n_outer, body, 0)

    return gather


def kernel(indexes, table, W):
    b, l = indexes.shape
    num, dim = table.shape
    P128 = _project_table(table, W)
    idx_flat = indexes.reshape(-1).astype(jnp.int32)
    return _make_gather(b, l, dim, 8)(P128, idx_flat)
